# Initial kernel scaffold; baseline (speedup 1.0000x reference)
import functools

import jax
import jax.numpy as jnp
from jax import lax
from jax.experimental import pallas as pl
from jax.experimental.pallas import tpu as pltpu
from jax.experimental.pallas import tpu_sc as plsc

VOCAB = 256
N_BLOCKS = 4
BLOCK_ROWS = 64
DIM = 64
B, L = 1024, 20
N_TOK = B * L

_info = plsc.get_sparse_core_info()
_NC, _NS, _LANES = _info.num_cores, _info.num_subcores, _info.num_lanes
_NW = _NC * _NS
_TOK_PER_W = N_TOK // _NW


def _make_sc_kernel():
    mesh = plsc.VectorSubcoreMesh(core_axis_name="c", subcore_axis_name="s")

    @functools.partial(
        pl.kernel,
        mesh=mesh,
        out_type=jax.ShapeDtypeStruct((N_TOK, DIM), jnp.float32),
        scratch_types=[
            pltpu.VMEM((_TOK_PER_W,), jnp.int32),
            pltpu.VMEM((VOCAB,), jnp.int32),
            pltpu.VMEM((VOCAB,), jnp.int32),
            pltpu.VMEM((_TOK_PER_W, DIM), jnp.float32),
            pltpu.SemaphoreType.DMA,
        ],
    )
    def sc_kernel(src_hbm, ba_hbm, la_hbm, table_hbm, out_hbm,
                  idx_v, ba_v, la_v, rows_v, sem):
        wid = lax.axis_index("s") * _NC + lax.axis_index("c")
        base = wid * _TOK_PER_W
        pltpu.sync_copy(ba_hbm, ba_v)
        pltpu.sync_copy(la_hbm, la_v)
        pltpu.sync_copy(src_hbm.at[pl.ds(base, _TOK_PER_W)], idx_v)

        def body(i, carry):
            sl = pl.ds(i * _LANES, _LANES)
            tok = idx_v[sl]
            blk = plsc.load_gather(ba_v, [tok])
            loc = plsc.load_gather(la_v, [tok])
            idx_v[sl] = blk * BLOCK_ROWS + loc
            return carry

        lax.fori_loop(0, _TOK_PER_W // _LANES, body, 0)
        pltpu.async_copy(table_hbm.at[idx_v], rows_v, sem).wait()
        pltpu.sync_copy(rows_v, out_hbm.at[pl.ds(base, _TOK_PER_W)])

    return sc_kernel


_sc_kernel = _make_sc_kernel()


def kernel(src, block_assign, local_assign, W0, W1, W2, W3):
    table = jnp.concatenate([W0, W1, W2, W3], axis=0)
    flat_src = src.reshape(N_TOK)
    out = _sc_kernel(flat_src, block_assign, local_assign, table)
    return out.reshape(B, L, DIM)


# speedup vs baseline: 4.7813x; 4.7813x over previous
import functools

import jax
import jax.numpy as jnp
from jax import lax
from jax.experimental import pallas as pl
from jax.experimental.pallas import tpu as pltpu
from jax.experimental.pallas import tpu_sc as plsc

VOCAB = 256
N_BLOCKS = 4
BLOCK_ROWS = 64
DIM = 64
B, L = 1024, 20
N_TOK = B * L

_info = plsc.get_sparse_core_info()
_NC, _NS, _LANES = _info.num_cores, _info.num_subcores, _info.num_lanes
_NW = _NC * _NS
_TOK_PER_W = N_TOK // _NW


def _make_sc_kernel():
    mesh = plsc.VectorSubcoreMesh(core_axis_name="c", subcore_axis_name="s")

    @functools.partial(
        pl.kernel,
        mesh=mesh,
        out_type=jax.ShapeDtypeStruct((N_TOK, DIM), jnp.float32),
        compiler_params=pltpu.CompilerParams(use_tc_tiling_on_sc=False),
        scratch_types=[
            pltpu.VMEM((_TOK_PER_W,), jnp.int32),
            pltpu.VMEM((_TOK_PER_W,), jnp.int32),
            pltpu.VMEM((_TOK_PER_W,), jnp.int32),
            pltpu.VMEM((_TOK_PER_W, DIM), jnp.float32),
            pltpu.SemaphoreType.DMA,
        ],
    )
    def sc_kernel(src_hbm, ba_hbm, la_hbm, table_hbm, out_hbm,
                  idx_v, blk_v, loc_v, rows_v, sem):
        wid = lax.axis_index("s") * _NC + lax.axis_index("c")
        base = wid * _TOK_PER_W
        pltpu.sync_copy(src_hbm.at[pl.ds(base, _TOK_PER_W)], idx_v)
        pltpu.async_copy(ba_hbm.at[idx_v], blk_v, sem).wait()
        pltpu.async_copy(la_hbm.at[idx_v], loc_v, sem).wait()

        def body(i, carry):
            sl = pl.ds(i * _LANES, _LANES)
            idx_v[sl] = blk_v[sl] * BLOCK_ROWS + loc_v[sl]
            return carry

        lax.fori_loop(0, _TOK_PER_W // _LANES, body, 0)
        pltpu.async_copy(table_hbm.at[idx_v], rows_v, sem).wait()
        pltpu.sync_copy(rows_v, out_hbm.at[pl.ds(base, _TOK_PER_W)])

    return sc_kernel


_sc_kernel = _make_sc_kernel()


def kernel(src, block_assign, local_assign, W0, W1, W2, W3):
    table = jnp.concatenate([W0, W1, W2, W3], axis=0)
    flat_src = src.reshape(N_TOK)
    out = _sc_kernel(flat_src, block_assign, local_assign, table)
    return out.reshape(B, L, DIM)


# speedup vs baseline: 6.9719x; 1.4582x over previous
import functools

import jax
import jax.numpy as jnp
from jax import lax
from jax.experimental import pallas as pl
from jax.experimental.pallas import tpu as pltpu
from jax.experimental.pallas import tpu_sc as plsc

VOCAB = 256
N_BLOCKS = 4
BLOCK_ROWS = 64
DIM = 64
B, L = 1024, 20
N_TOK = B * L

_info = plsc.get_sparse_core_info()
_NC, _NS, _LANES = _info.num_cores, _info.num_subcores, _info.num_lanes
_NW = _NC * _NS
_TOK_PER_W = N_TOK // _NW


def _make_sc_kernel():
    mesh = plsc.VectorSubcoreMesh(core_axis_name="c", subcore_axis_name="s")

    @functools.partial(
        pl.kernel,
        mesh=mesh,
        out_type=jax.ShapeDtypeStruct((N_TOK, DIM), jnp.float32),
        compiler_params=pltpu.CompilerParams(use_tc_tiling_on_sc=False),
        scratch_types=[
            pltpu.VMEM((_TOK_PER_W,), jnp.int32),
            pltpu.VMEM((_TOK_PER_W,), jnp.int32),
            pltpu.VMEM((_TOK_PER_W, DIM), jnp.float32),
            pltpu.SemaphoreType.DMA,
        ],
    )
    def sc_kernel(src_hbm, row_map_hbm, table_hbm, out_hbm,
                  idx_v, row_v, rows_v, sem):
        wid = lax.axis_index("s") * _NC + lax.axis_index("c")
        base = wid * _TOK_PER_W
        pltpu.sync_copy(src_hbm.at[pl.ds(base, _TOK_PER_W)], idx_v)
        pltpu.async_copy(row_map_hbm.at[idx_v], row_v, sem).wait()
        pltpu.async_copy(table_hbm.at[row_v], rows_v, sem).wait()
        pltpu.sync_copy(rows_v, out_hbm.at[pl.ds(base, _TOK_PER_W)])

    return sc_kernel


_sc_kernel = _make_sc_kernel()


def kernel(src, block_assign, local_assign, W0, W1, W2, W3):
    table = jnp.concatenate([W0, W1, W2, W3], axis=0)
    row_map = block_assign * BLOCK_ROWS + local_assign
    flat_src = src.reshape(N_TOK)
    out = _sc_kernel(flat_src, row_map, table)
    return out.reshape(B, L, DIM)


# speedup vs baseline: 13.2204x; 1.8962x over previous
import functools

import jax
import jax.numpy as jnp
from jax import lax
from jax.experimental import pallas as pl
from jax.experimental.pallas import tpu as pltpu
from jax.experimental.pallas import tpu_sc as plsc

VOCAB = 256
N_BLOCKS = 4
BLOCK_ROWS = 64
DIM = 64
B, L = 1024, 20
N_TOK = B * L

_info = plsc.get_sparse_core_info()
_NC, _NS, _LANES = _info.num_cores, _info.num_subcores, _info.num_lanes
_NW = _NC * _NS
_TOK_PER_W = N_TOK // _NW


def _make_sc_kernel():
    mesh = plsc.VectorSubcoreMesh(core_axis_name="c", subcore_axis_name="s")

    @functools.partial(
        pl.kernel,
        mesh=mesh,
        out_type=jax.ShapeDtypeStruct((N_TOK, DIM), jnp.float32),
        compiler_params=pltpu.CompilerParams(use_tc_tiling_on_sc=False),
        scratch_types=[
            pltpu.VMEM((_TOK_PER_W,), jnp.int32),
            pltpu.VMEM((_TOK_PER_W, DIM), jnp.float32),
            pltpu.SemaphoreType.DMA,
        ],
    )
    def sc_kernel(src_hbm, table_hbm, out_hbm, idx_v, rows_v, sem):
        wid = lax.axis_index("s") * _NC + lax.axis_index("c")
        base = wid * _TOK_PER_W
        pltpu.sync_copy(src_hbm.at[pl.ds(base, _TOK_PER_W)], idx_v)
        pltpu.async_copy(table_hbm.at[idx_v], rows_v, sem).wait()
        pltpu.sync_copy(rows_v, out_hbm.at[pl.ds(base, _TOK_PER_W)])

    return sc_kernel


_sc_kernel = _make_sc_kernel()


def kernel(src, block_assign, local_assign, W0, W1, W2, W3):
    table = jnp.concatenate([W0, W1, W2, W3], axis=0)
    row_map = block_assign * BLOCK_ROWS + local_assign
    table = jnp.take(table, row_map, axis=0)
    flat_src = src.reshape(N_TOK)
    out = _sc_kernel(flat_src, table)
    return out.reshape(B, L, DIM)
